# baseline (speedup 1.0000x reference)
import functools

import jax
import jax.numpy as jnp
from jax import lax
from jax.experimental import pallas as pl
from jax.experimental.pallas import tpu as pltpu
from jax.experimental.pallas import tpu_sc as plsc

N = 10000
E = 320000
D = 128
H = 128
C = 6
NID = 1000

NC, NS, L = 2, 16, 16
NW = NC * NS
EPT = E // NW
K = 40
NCH = EPT // K
NPAD = 10240
NPT = NPAD // NS
NID_PAD = 1024
IPT = NID_PAD // NW

_mesh = plsc.VectorSubcoreMesh(
    core_axis_name="c", subcore_axis_name="s", num_cores=NC, num_subcores=NS
)


@functools.partial(
    pl.kernel,
    out_type=[
        jax.ShapeDtypeStruct((NC, NPAD, D), jnp.float32),
        jax.ShapeDtypeStruct((NC, NPAD, D), jnp.float32),
    ],
    mesh=_mesh,
    scratch_types=[
        pltpu.VMEM((2, K), jnp.int32),
        pltpu.VMEM((K, L), jnp.float32),
        pltpu.VMEM((K, D), jnp.float32),
        pltpu.VMEM((1, IPT), jnp.int32),
        pltpu.VMEM((IPT, L), jnp.float32),
        pltpu.VMEM((16, D), jnp.float32),
        pltpu.VMEM_SHARED((NPAD, D), jnp.float32),
    ],
)
def _sc_precompute(rc4, ewb4, id3, idvb3, deg_out, mask_out,
                   rc_v, ewb_v, wide_v, id_v, idvb_v, zbuf, tab_sh):
    c = lax.axis_index("c")
    sid = lax.axis_index("s")
    wid = c * NS + sid
    for r in range(16):
        for j in range(D // L):
            zbuf[r, pl.ds(j * L, L)] = jnp.zeros((L,), jnp.float32)

    def zc(z, carry):
        pltpu.sync_copy(zbuf, tab_sh.at[pl.ds(sid * NPT + z * 16, 16)])
        return carry

    lax.fori_loop(0, NPT // 16, zc, 0)
    pltpu.sync_copy(id3.at[wid], id_v)
    pltpu.sync_copy(idvb3.at[wid], idvb_v)
    plsc.subcore_barrier()

    def chunk(g, carry):
        pltpu.sync_copy(rc4.at[wid, g], rc_v)
        pltpu.sync_copy(ewb4.at[wid, g], ewb_v)
        for e in range(K):
            w = ewb_v[e, :]
            for j in range(D // L):
                wide_v[e, pl.ds(j * L, L)] = w
        pltpu.sync_copy(wide_v, tab_sh.at[rc_v.at[0]], add=True)
        return carry

    lax.fori_loop(0, NCH, chunk, 0)
    plsc.subcore_barrier()
    pltpu.sync_copy(tab_sh.at[pl.ds(sid * NPT, NPT)],
                    deg_out.at[c, pl.ds(sid * NPT, NPT)])
    plsc.subcore_barrier()

    lax.fori_loop(0, NPT // 16, zc, 0)
    plsc.subcore_barrier()
    for e in range(IPT):
        w = idvb_v[e, :]
        for j in range(D // L):
            wide_v[e, pl.ds(j * L, L)] = w
    pltpu.sync_copy(wide_v.at[pl.ds(0, IPT)], tab_sh.at[id_v.at[0]], add=True)
    plsc.subcore_barrier()
    pltpu.sync_copy(tab_sh.at[pl.ds(sid * NPT, NPT)],
                    mask_out.at[c, pl.ds(sid * NPT, NPT)])


@functools.partial(
    pl.kernel,
    out_type=jax.ShapeDtypeStruct((NC, NPAD, D), jnp.float32),
    mesh=_mesh,
    scratch_types=(
        [pltpu.VMEM((2, K), jnp.int32) for _ in range(5)]
        + [pltpu.VMEM((K, D), jnp.float32) for _ in range(5)]
        + [
            pltpu.VMEM((K, L), jnp.float32),
            pltpu.VMEM((16, D), jnp.float32),
            pltpu.VMEM_SHARED((NPAD, D), jnp.float32),
        ]
        + [pltpu.SemaphoreType.DMA for _ in range(10)]
    ),
)
def _sc_agg(s_hbm, rc4, ewb4, p_out, *refs):
    rc = refs[0:5]
    buf = refs[5:10]
    ewb_v, zbuf, agg_sh = refs[10:13]
    gsem = refs[13:18]
    ssem = refs[18:23]
    c = lax.axis_index("c")
    sid = lax.axis_index("s")
    wid = c * NS + sid
    for r in range(16):
        for j in range(D // L):
            zbuf[r, pl.ds(j * L, L)] = jnp.zeros((L,), jnp.float32)

    def zc(z, carry):
        pltpu.sync_copy(zbuf, agg_sh.at[pl.ds(sid * NPT + z * 16, 16)])
        return carry

    lax.fori_loop(0, NPT // 16, zc, 0)
    plsc.subcore_barrier()

    def issue(g, s):
        pltpu.sync_copy(rc4.at[wid, g], rc[s])
        pltpu.async_copy(s_hbm.at[rc[s].at[1]], buf[s], gsem[s])

    def drain(s):
        pltpu.make_async_copy(
            buf[s], agg_sh.at[rc[s].at[0]], ssem[s]).wait()

    def work_fire(g, s):
        pltpu.make_async_copy(s_hbm.at[rc[s].at[1]], buf[s], gsem[s]).wait()
        pltpu.sync_copy(ewb4.at[wid, g], ewb_v)
        b = buf[s]
        for e in range(K):
            w = ewb_v[e, :]
            for j in range(D // L):
                b[e, pl.ds(j * L, L)] = b[e, pl.ds(j * L, L)] * w
        pltpu.async_copy(b, agg_sh.at[rc[s].at[0]], ssem[s], add=True)

    for g0 in range(4):
        issue(g0, g0)

    def pipe(i, carry):
        for b in range(5):
            g = 5 * i + b
            sp = (b - 1) % 5
            if b == 0:
                @pl.when(i >= 1)
                def _():
                    drain(sp)
            else:
                drain(sp)
            work_fire(g, b)
            if b == 0:
                issue(g + 4, (b + 4) % 5)
            else:
                @pl.when(i <= NCH // 5 - 2)
                def _():
                    issue(g + 4, (b + 4) % 5)
        return carry

    lax.fori_loop(0, NCH // 5, pipe, 0)
    drain((NCH - 1) % 5)
    plsc.subcore_barrier()
    pltpu.sync_copy(agg_sh.at[pl.ds(sid * NPT, NPT)],
                    p_out.at[c, pl.ds(sid * NPT, NPT)])


BLK = 2000
NB = N // BLK


def _dinv_of(dp_ref):
    deg = dp_ref[0, :, 0] + dp_ref[1, :, 0]
    return jnp.where(deg > 0.0, lax.rsqrt(jnp.maximum(deg, 1e-12)), 0.0)


def _mask_of(mp_ref):
    return (mp_ref[0, :, 0] + mp_ref[1, :, 0]) > 0.0


def _tc_layer0_body(x_ref, wc_ref, dp_ref, mp_ref, o_ref):
    dinv = _dinv_of(dp_ref)
    m = _mask_of(mp_ref)
    t = jnp.dot(x_ref[...], wc_ref[...], preferred_element_type=jnp.float32)
    sel = jnp.where(m[:, None], t[:, D:], t[:, :D])
    o_ref[...] = sel * dinv[:, None]


def _tc_act_body(p_ref, wc_ref, dp_ref, mp_ref, b_ref, o_ref):
    dinv = _dinv_of(dp_ref)
    m = _mask_of(mp_ref)
    h = jax.nn.relu((p_ref[0] + p_ref[1]) * dinv[:, None] + b_ref[...])
    t = jnp.dot(h, wc_ref[...], preferred_element_type=jnp.float32)
    sel = jnp.where(m[:, None], t[:, D:], t[:, :D])
    o_ref[...] = sel * dinv[:, None]


def _tc_head_body(p_ref, dp_ref, b2_ref, wm1_ref, bm1_ref, wm2_ref, bm2_ref,
                  o_ref):
    dinv = _dinv_of(dp_ref)
    h = jax.nn.relu((p_ref[0] + p_ref[1]) * dinv[:, None] + b2_ref[...])
    z = jax.nn.relu(
        jnp.dot(h, wm1_ref[...], preferred_element_type=jnp.float32)
        + bm1_ref[...])
    o_ref[...] = (
        jnp.dot(z, wm2_ref[...], preferred_element_type=jnp.float32)
        + bm2_ref[...])


_spec_dp = pl.BlockSpec((2, BLK, D), lambda i: (0, i, 0))
_spec_rows = pl.BlockSpec((BLK, D), lambda i: (i, 0))
_spec_p = pl.BlockSpec((2, BLK, D), lambda i: (0, i, 0))


def _tc_layer0(x, wc, dp, mp):
    return pl.pallas_call(
        _tc_layer0_body,
        grid=(NB,),
        in_specs=[
            _spec_rows,
            pl.BlockSpec((D, 2 * D), lambda i: (0, 0)),
            _spec_dp,
            _spec_dp,
        ],
        out_specs=_spec_rows,
        out_shape=jax.ShapeDtypeStruct((N, D), jnp.float32),
    )(x, wc, dp, mp)


def _tc_act(p, wc, dp, mp, b):
    return pl.pallas_call(
        _tc_act_body,
        grid=(NB,),
        in_specs=[
            _spec_p,
            pl.BlockSpec((D, 2 * D), lambda i: (0, 0)),
            _spec_dp,
            _spec_dp,
            pl.BlockSpec((1, D), lambda i: (0, 0)),
        ],
        out_specs=_spec_rows,
        out_shape=jax.ShapeDtypeStruct((N, D), jnp.float32),
    )(p, wc, dp, mp, b)


def _tc_head(p, dp, b2, wm1, bm1, wm2p, bm2p):
    return pl.pallas_call(
        _tc_head_body,
        grid=(NB,),
        in_specs=[
            _spec_p,
            _spec_dp,
            pl.BlockSpec((1, D), lambda i: (0, 0)),
            pl.BlockSpec((D, 256), lambda i: (0, 0)),
            pl.BlockSpec((1, 256), lambda i: (0, 0)),
            pl.BlockSpec((256, 128), lambda i: (0, 0)),
            pl.BlockSpec((1, 128), lambda i: (0, 0)),
        ],
        out_specs=_spec_rows,
        out_shape=jax.ShapeDtypeStruct((N, D), jnp.float32),
    )(p, dp, b2, wm1, bm1, wm2p, bm2p)


def kernel(x, edge_index, id_index, edge_weight, W0, IW0, b0, W1, IW1, b1,
           W2, IW2, b2, Wm1, bm1, Wm2, bm2):
    rc4 = edge_index.reshape(2, NW, NCH, K).transpose(1, 2, 0, 3)
    ewb4 = jnp.broadcast_to(edge_weight[:, None], (E, L)).reshape(NW, NCH, K, L)
    idp = jnp.concatenate(
        [id_index, jnp.zeros((NID_PAD - NID,), jnp.int32)]).reshape(NW, 1, IPT)
    idvb3 = jnp.broadcast_to(
        ((jnp.arange(NID_PAD) < NID).astype(jnp.float32))[:, None],
        (NID_PAD, L)).reshape(NW, IPT, L)

    deg_p, mask_p = _sc_precompute(rc4, ewb4, idp, idvb3)

    wc0 = jnp.concatenate([W0, IW0], axis=1)
    wc1 = jnp.concatenate([W1, IW1], axis=1)
    wc2 = jnp.concatenate([W2, IW2], axis=1)
    b0r = b0.reshape(1, D)
    b1r = b1.reshape(1, D)
    b2r = b2.reshape(1, D)
    bm1r = bm1.reshape(1, 256)
    wm2p = jnp.zeros((256, 128), jnp.float32).at[:, :C].set(Wm2)
    bm2p = jnp.zeros((1, 128), jnp.float32).at[0, :C].set(bm2)

    s0 = _tc_layer0(x, wc0, deg_p, mask_p)
    p1 = _sc_agg(s0, rc4, ewb4)
    s1 = _tc_act(p1, wc1, deg_p, mask_p, b0r)
    p2 = _sc_agg(s1, rc4, ewb4)
    s2 = _tc_act(p2, wc2, deg_p, mask_p, b1r)
    p3 = _sc_agg(s2, rc4, ewb4)
    outp = _tc_head(p3, deg_p, b2r, Wm1, bm1r, wm2p, bm2p)
    return outp[:, :C]


# speedup vs baseline: 1.2704x; 1.2704x over previous
import functools

import jax
import jax.numpy as jnp
from jax import lax
from jax.experimental import pallas as pl
from jax.experimental.pallas import tpu as pltpu
from jax.experimental.pallas import tpu_sc as plsc

N = 10000
E = 320000
D = 128
H = 128
C = 6
NID = 1000

NC, NS, L = 2, 16, 16
NW = NC * NS
EPT = E // NW
K = 80
NCH = EPT // K
NPAD = 10240
NPT = NPAD // NS
NID_PAD = 1024
IPT = NID_PAD // NW

_mesh = plsc.VectorSubcoreMesh(
    core_axis_name="c", subcore_axis_name="s", num_cores=NC, num_subcores=NS
)


@functools.partial(
    pl.kernel,
    out_type=[
        jax.ShapeDtypeStruct((NC, NPAD, D), jnp.float32),
        jax.ShapeDtypeStruct((NC, NPAD, D), jnp.float32),
    ],
    mesh=_mesh,
    scratch_types=[
        pltpu.VMEM((2, K), jnp.int32),
        pltpu.VMEM((K, L), jnp.float32),
        pltpu.VMEM((K, D), jnp.float32),
        pltpu.VMEM((1, IPT), jnp.int32),
        pltpu.VMEM((IPT, L), jnp.float32),
        pltpu.VMEM((16, D), jnp.float32),
        pltpu.VMEM_SHARED((NPAD, D), jnp.float32),
    ],
)
def _sc_precompute(rc4, ewb4, id3, idvb3, deg_out, mask_out,
                   rc_v, ewb_v, wide_v, id_v, idvb_v, zbuf, tab_sh):
    c = lax.axis_index("c")
    sid = lax.axis_index("s")
    wid = c * NS + sid
    for r in range(16):
        for j in range(D // L):
            zbuf[r, pl.ds(j * L, L)] = jnp.zeros((L,), jnp.float32)

    def zc(z, carry):
        pltpu.sync_copy(zbuf, tab_sh.at[pl.ds(sid * NPT + z * 16, 16)])
        return carry

    lax.fori_loop(0, NPT // 16, zc, 0)
    pltpu.sync_copy(id3.at[wid], id_v)
    pltpu.sync_copy(idvb3.at[wid], idvb_v)
    plsc.subcore_barrier()

    def chunk(g, carry):
        pltpu.sync_copy(rc4.at[wid, g], rc_v)
        pltpu.sync_copy(ewb4.at[wid, g], ewb_v)
        for e in range(K):
            w = ewb_v[e, :]
            for j in range(D // L):
                wide_v[e, pl.ds(j * L, L)] = w
        pltpu.sync_copy(wide_v, tab_sh.at[rc_v.at[0]], add=True)
        return carry

    lax.fori_loop(0, NCH, chunk, 0)
    plsc.subcore_barrier()
    pltpu.sync_copy(tab_sh.at[pl.ds(sid * NPT, NPT)],
                    deg_out.at[c, pl.ds(sid * NPT, NPT)])
    plsc.subcore_barrier()

    lax.fori_loop(0, NPT // 16, zc, 0)
    plsc.subcore_barrier()
    for e in range(IPT):
        w = idvb_v[e, :]
        for j in range(D // L):
            wide_v[e, pl.ds(j * L, L)] = w
    pltpu.sync_copy(wide_v.at[pl.ds(0, IPT)], tab_sh.at[id_v.at[0]], add=True)
    plsc.subcore_barrier()
    pltpu.sync_copy(tab_sh.at[pl.ds(sid * NPT, NPT)],
                    mask_out.at[c, pl.ds(sid * NPT, NPT)])


@functools.partial(
    pl.kernel,
    out_type=jax.ShapeDtypeStruct((NC, NPAD, D), jnp.float32),
    mesh=_mesh,
    scratch_types=(
        [pltpu.VMEM((2, K), jnp.int32) for _ in range(3)]
        + [pltpu.VMEM((K, D), jnp.float32) for _ in range(3)]
        + [
            pltpu.VMEM((K, L), jnp.float32),
            pltpu.VMEM((8, D), jnp.float32),
            pltpu.VMEM_SHARED((NPAD, D), jnp.float32),
        ]
        + [pltpu.SemaphoreType.DMA for _ in range(6)]
    ),
)
def _sc_agg(s_hbm, rc4, ewb4, p_out, *refs):
    rc = refs[0:3]
    buf = refs[3:6]
    ewb_v, zbuf, agg_sh = refs[6:9]
    gsem = refs[9:12]
    ssem = refs[12:15]
    c = lax.axis_index("c")
    sid = lax.axis_index("s")
    wid = c * NS + sid
    for r in range(8):
        for j in range(D // L):
            zbuf[r, pl.ds(j * L, L)] = jnp.zeros((L,), jnp.float32)

    def zc(z, carry):
        pltpu.sync_copy(zbuf, agg_sh.at[pl.ds(sid * NPT + z * 8, 8)])
        return carry

    lax.fori_loop(0, NPT // 8, zc, 0)
    plsc.subcore_barrier()

    def issue(g, s):
        pltpu.sync_copy(rc4.at[wid, g], rc[s])
        pltpu.async_copy(s_hbm.at[rc[s].at[1]], buf[s], gsem[s])

    def drain(s):
        pltpu.make_async_copy(
            buf[s], agg_sh.at[rc[s].at[0]], ssem[s]).wait()

    def work_fire(g, s):
        pltpu.make_async_copy(s_hbm.at[rc[s].at[1]], buf[s], gsem[s]).wait()
        pltpu.sync_copy(ewb4.at[wid, g], ewb_v)
        b = buf[s]
        for e in range(K):
            w = ewb_v[e, :]
            for j in range(D // L):
                b[e, pl.ds(j * L, L)] = b[e, pl.ds(j * L, L)] * w
        pltpu.async_copy(b, agg_sh.at[rc[s].at[0]], ssem[s], add=True)

    issue(0, 0)
    issue(1, 1)

    def pipe(i, carry):
        for b in range(3):
            g = 3 * i + b
            sp = (b - 1) % 3
            if b == 0:
                @pl.when(i >= 1)
                def _():
                    drain(sp)
            else:
                drain(sp)
            work_fire(g, b)
            issue(g + 2, (b + 2) % 3)
        return carry

    lax.fori_loop(0, (NCH - 2) // 3, pipe, 0)
    drain(2)
    work_fire(NCH - 2, 0)
    work_fire(NCH - 1, 1)
    drain(0)
    drain(1)
    plsc.subcore_barrier()
    pltpu.sync_copy(agg_sh.at[pl.ds(sid * NPT, NPT)],
                    p_out.at[c, pl.ds(sid * NPT, NPT)])


BLK = 2000
NB = N // BLK


def _dinv_of(dp_ref):
    deg = dp_ref[0, :, 0] + dp_ref[1, :, 0]
    return jnp.where(deg > 0.0, lax.rsqrt(jnp.maximum(deg, 1e-12)), 0.0)


def _mask_of(mp_ref):
    return (mp_ref[0, :, 0] + mp_ref[1, :, 0]) > 0.0


def _tc_layer0_body(x_ref, wc_ref, dp_ref, mp_ref, o_ref):
    dinv = _dinv_of(dp_ref)
    m = _mask_of(mp_ref)
    t = jnp.dot(x_ref[...], wc_ref[...], preferred_element_type=jnp.float32)
    sel = jnp.where(m[:, None], t[:, D:], t[:, :D])
    o_ref[...] = sel * dinv[:, None]


def _tc_act_body(p_ref, wc_ref, dp_ref, mp_ref, b_ref, o_ref):
    dinv = _dinv_of(dp_ref)
    m = _mask_of(mp_ref)
    h = jax.nn.relu((p_ref[0] + p_ref[1]) * dinv[:, None] + b_ref[...])
    t = jnp.dot(h, wc_ref[...], preferred_element_type=jnp.float32)
    sel = jnp.where(m[:, None], t[:, D:], t[:, :D])
    o_ref[...] = sel * dinv[:, None]


def _tc_head_body(p_ref, dp_ref, b2_ref, wm1_ref, bm1_ref, wm2_ref, bm2_ref,
                  o_ref):
    dinv = _dinv_of(dp_ref)
    h = jax.nn.relu((p_ref[0] + p_ref[1]) * dinv[:, None] + b2_ref[...])
    z = jax.nn.relu(
        jnp.dot(h, wm1_ref[...], preferred_element_type=jnp.float32)
        + bm1_ref[...])
    o_ref[...] = (
        jnp.dot(z, wm2_ref[...], preferred_element_type=jnp.float32)
        + bm2_ref[...])


_spec_dp = pl.BlockSpec((2, BLK, D), lambda i: (0, i, 0))
_spec_rows = pl.BlockSpec((BLK, D), lambda i: (i, 0))
_spec_p = pl.BlockSpec((2, BLK, D), lambda i: (0, i, 0))


def _tc_layer0(x, wc, dp, mp):
    return pl.pallas_call(
        _tc_layer0_body,
        grid=(NB,),
        in_specs=[
            _spec_rows,
            pl.BlockSpec((D, 2 * D), lambda i: (0, 0)),
            _spec_dp,
            _spec_dp,
        ],
        out_specs=_spec_rows,
        out_shape=jax.ShapeDtypeStruct((N, D), jnp.float32),
    )(x, wc, dp, mp)


def _tc_act(p, wc, dp, mp, b):
    return pl.pallas_call(
        _tc_act_body,
        grid=(NB,),
        in_specs=[
            _spec_p,
            pl.BlockSpec((D, 2 * D), lambda i: (0, 0)),
            _spec_dp,
            _spec_dp,
            pl.BlockSpec((1, D), lambda i: (0, 0)),
        ],
        out_specs=_spec_rows,
        out_shape=jax.ShapeDtypeStruct((N, D), jnp.float32),
    )(p, wc, dp, mp, b)


def _tc_head(p, dp, b2, wm1, bm1, wm2p, bm2p):
    return pl.pallas_call(
        _tc_head_body,
        grid=(NB,),
        in_specs=[
            _spec_p,
            _spec_dp,
            pl.BlockSpec((1, D), lambda i: (0, 0)),
            pl.BlockSpec((D, 256), lambda i: (0, 0)),
            pl.BlockSpec((1, 256), lambda i: (0, 0)),
            pl.BlockSpec((256, 128), lambda i: (0, 0)),
            pl.BlockSpec((1, 128), lambda i: (0, 0)),
        ],
        out_specs=_spec_rows,
        out_shape=jax.ShapeDtypeStruct((N, D), jnp.float32),
    )(p, dp, b2, wm1, bm1, wm2p, bm2p)


def kernel(x, edge_index, id_index, edge_weight, W0, IW0, b0, W1, IW1, b1,
           W2, IW2, b2, Wm1, bm1, Wm2, bm2):
    rc4 = edge_index.reshape(2, NW, NCH, K).transpose(1, 2, 0, 3)
    ewb4 = jnp.broadcast_to(edge_weight[:, None], (E, L)).reshape(NW, NCH, K, L)
    idp = jnp.concatenate(
        [id_index, jnp.zeros((NID_PAD - NID,), jnp.int32)]).reshape(NW, 1, IPT)
    idvb3 = jnp.broadcast_to(
        ((jnp.arange(NID_PAD) < NID).astype(jnp.float32))[:, None],
        (NID_PAD, L)).reshape(NW, IPT, L)

    deg_p, mask_p = _sc_precompute(rc4, ewb4, idp, idvb3)

    wc0 = jnp.concatenate([W0, IW0], axis=1)
    wc1 = jnp.concatenate([W1, IW1], axis=1)
    wc2 = jnp.concatenate([W2, IW2], axis=1)
    b0r = b0.reshape(1, D)
    b1r = b1.reshape(1, D)
    b2r = b2.reshape(1, D)
    bm1r = bm1.reshape(1, 256)
    wm2p = jnp.zeros((256, 128), jnp.float32).at[:, :C].set(Wm2)
    bm2p = jnp.zeros((1, 128), jnp.float32).at[0, :C].set(bm2)

    s0 = _tc_layer0(x, wc0, deg_p, mask_p)
    p1 = _sc_agg(s0, rc4, ewb4)
    s1 = _tc_act(p1, wc1, deg_p, mask_p, b0r)
    p2 = _sc_agg(s1, rc4, ewb4)
    s2 = _tc_act(p2, wc2, deg_p, mask_p, b1r)
    p3 = _sc_agg(s2, rc4, ewb4)
    outp = _tc_head(p3, deg_p, b2r, Wm1, bm1r, wm2p, bm2p)
    return outp[:, :C]


# speedup vs baseline: 1.6125x; 1.2693x over previous
import functools

import jax
import jax.numpy as jnp
from jax import lax
from jax.experimental import pallas as pl
from jax.experimental.pallas import tpu as pltpu
from jax.experimental.pallas import tpu_sc as plsc

N = 10000
E = 320000
D = 128
H = 128
C = 6
NID = 1000

NC, NS, L = 2, 16, 16
NW = NC * NS
EPT = E // NW
K = 80
NCH = EPT // K
NPAD = 10240
NPT = NPAD // NS
NID_PAD = 1024
IPT = NID_PAD // NW

_mesh = plsc.VectorSubcoreMesh(
    core_axis_name="c", subcore_axis_name="s", num_cores=NC, num_subcores=NS
)


@functools.partial(
    pl.kernel,
    out_type=[
        jax.ShapeDtypeStruct((NC, NPAD, D), jnp.float32),
        jax.ShapeDtypeStruct((NC, NPAD, D), jnp.float32),
    ],
    mesh=_mesh,
    scratch_types=[
        pltpu.VMEM((2, K), jnp.int32),
        pltpu.VMEM((K, L), jnp.float32),
        pltpu.VMEM((K, D), jnp.float32),
        pltpu.VMEM((1, IPT), jnp.int32),
        pltpu.VMEM((IPT, L), jnp.float32),
        pltpu.VMEM((16, D), jnp.float32),
        pltpu.VMEM_SHARED((NPAD, D), jnp.float32),
    ],
)
def _sc_precompute(rc4, ewb4, id3, idvb3, deg_out, mask_out,
                   rc_v, ewb_v, wide_v, id_v, idvb_v, zbuf, tab_sh):
    c = lax.axis_index("c")
    sid = lax.axis_index("s")
    wid = c * NS + sid
    for r in range(16):
        for j in range(D // L):
            zbuf[r, pl.ds(j * L, L)] = jnp.zeros((L,), jnp.float32)

    def zc(z, carry):
        pltpu.sync_copy(zbuf, tab_sh.at[pl.ds(sid * NPT + z * 16, 16)])
        return carry

    lax.fori_loop(0, NPT // 16, zc, 0)
    pltpu.sync_copy(id3.at[wid], id_v)
    pltpu.sync_copy(idvb3.at[wid], idvb_v)
    plsc.subcore_barrier()

    def chunk(g, carry):
        pltpu.sync_copy(rc4.at[wid, g], rc_v)
        pltpu.sync_copy(ewb4.at[wid, g], ewb_v)
        for e in range(K):
            w = ewb_v[e, :]
            for j in range(D // L):
                wide_v[e, pl.ds(j * L, L)] = w
        pltpu.sync_copy(wide_v, tab_sh.at[rc_v.at[0]], add=True)
        return carry

    lax.fori_loop(0, NCH, chunk, 0)
    plsc.subcore_barrier()
    pltpu.sync_copy(tab_sh.at[pl.ds(sid * NPT, NPT)],
                    deg_out.at[c, pl.ds(sid * NPT, NPT)])
    plsc.subcore_barrier()

    lax.fori_loop(0, NPT // 16, zc, 0)
    plsc.subcore_barrier()
    for e in range(IPT):
        w = idvb_v[e, :]
        for j in range(D // L):
            wide_v[e, pl.ds(j * L, L)] = w
    pltpu.sync_copy(wide_v.at[pl.ds(0, IPT)], tab_sh.at[id_v.at[0]], add=True)
    plsc.subcore_barrier()
    pltpu.sync_copy(tab_sh.at[pl.ds(sid * NPT, NPT)],
                    mask_out.at[c, pl.ds(sid * NPT, NPT)])


@functools.partial(
    pl.kernel,
    out_type=jax.ShapeDtypeStruct((NC, NPAD, D), jnp.float32),
    mesh=_mesh,
    scratch_types=(
        [pltpu.VMEM((2, K), jnp.int32) for _ in range(3)]
        + [pltpu.VMEM((K, D), jnp.float32) for _ in range(3)]
        + [pltpu.VMEM((K * L // D, D), jnp.float32) for _ in range(3)]
        + [
            pltpu.VMEM((8, D), jnp.float32),
            pltpu.VMEM_SHARED((NPAD, D), jnp.float32),
        ]
        + [pltpu.SemaphoreType.DMA for _ in range(9)]
    ),
)
def _sc_agg(s_hbm, rc4, ewq4, p_out, *refs):
    rc = refs[0:3]
    buf = refs[3:6]
    ewq = refs[6:9]
    zbuf, agg_sh = refs[9:11]
    gsem = refs[11:14]
    ssem = refs[14:17]
    esem = refs[17:20]
    c = lax.axis_index("c")
    sid = lax.axis_index("s")
    wid = c * NS + sid
    for r in range(8):
        for j in range(D // L):
            zbuf[r, pl.ds(j * L, L)] = jnp.zeros((L,), jnp.float32)

    def zc(z, carry):
        pltpu.sync_copy(zbuf,
                        agg_sh.at[pl.ds(sid * NPT + z * 8, 8)])
        return carry

    lax.fori_loop(0, NPT // 8, zc, 0)
    plsc.subcore_barrier()

    def issue(g, s):
        pltpu.sync_copy(rc4.at[wid, g], rc[s])
        pltpu.async_copy(s_hbm.at[rc[s].at[1]], buf[s], gsem[s])
        pltpu.async_copy(ewq4.at[wid, g], ewq[s], esem[s])

    def drain(s):
        pltpu.make_async_copy(
            buf[s], agg_sh.at[rc[s].at[0]], ssem[s]).wait()

    def work_fire(g, s):
        pltpu.make_async_copy(s_hbm.at[rc[s].at[1]], buf[s], gsem[s]).wait()
        pltpu.make_async_copy(ewq4.at[wid, g], ewq[s], esem[s]).wait()
        b = buf[s]
        q = ewq[s]
        for e in range(K):
            w = q[e >> 3, pl.ds((e & 7) * L, L)]
            for j in range(D // L):
                b[e, pl.ds(j * L, L)] = b[e, pl.ds(j * L, L)] * w
        pltpu.async_copy(b, agg_sh.at[rc[s].at[0]], ssem[s], add=True)

    issue(0, 0)
    issue(1, 1)

    def pipe(i, carry):
        for b in range(3):
            g = 3 * i + b
            sp = (b - 1) % 3
            if b == 0:
                @pl.when(i >= 1)
                def _():
                    drain(sp)
            else:
                drain(sp)
            work_fire(g, b)
            issue(g + 2, (b + 2) % 3)
        return carry

    lax.fori_loop(0, (NCH - 2) // 3, pipe, 0)
    drain(2)
    work_fire(NCH - 2, 0)
    work_fire(NCH - 1, 1)
    drain(0)
    drain(1)
    plsc.subcore_barrier()
    pltpu.sync_copy(agg_sh.at[pl.ds(sid * NPT, NPT)],
                    p_out.at[c, pl.ds(sid * NPT, NPT)])


BLK = 2000
NB = N // BLK


def _dinv_of(dp_ref):
    deg = dp_ref[0, :, 0] + dp_ref[1, :, 0]
    return jnp.where(deg > 0.0, lax.rsqrt(jnp.maximum(deg, 1e-12)), 0.0)


def _mask_of(mp_ref):
    return (mp_ref[0, :, 0] + mp_ref[1, :, 0]) > 0.0


def _tc_layer0_body(x_ref, wc_ref, dp_ref, mp_ref, o_ref):
    dinv = _dinv_of(dp_ref)
    m = _mask_of(mp_ref)
    t = jnp.dot(x_ref[...], wc_ref[...], preferred_element_type=jnp.float32)
    sel = jnp.where(m[:, None], t[:, D:], t[:, :D])
    o_ref[...] = sel * dinv[:, None]


def _tc_act_body(p_ref, wc_ref, dp_ref, mp_ref, b_ref, o_ref):
    dinv = _dinv_of(dp_ref)
    m = _mask_of(mp_ref)
    h = jax.nn.relu((p_ref[0] + p_ref[1]) * dinv[:, None] + b_ref[...])
    t = jnp.dot(h, wc_ref[...], preferred_element_type=jnp.float32)
    sel = jnp.where(m[:, None], t[:, D:], t[:, :D])
    o_ref[...] = sel * dinv[:, None]


def _tc_head_body(p_ref, dp_ref, b2_ref, wm1_ref, bm1_ref, wm2_ref, bm2_ref,
                  o_ref):
    dinv = _dinv_of(dp_ref)
    h = jax.nn.relu((p_ref[0] + p_ref[1]) * dinv[:, None] + b2_ref[...])
    z = jax.nn.relu(
        jnp.dot(h, wm1_ref[...], preferred_element_type=jnp.float32)
        + bm1_ref[...])
    o_ref[...] = (
        jnp.dot(z, wm2_ref[...], preferred_element_type=jnp.float32)
        + bm2_ref[...])


_spec_dp = pl.BlockSpec((2, BLK, D), lambda i: (0, i, 0))
_spec_rows = pl.BlockSpec((BLK, D), lambda i: (i, 0))
_spec_p = pl.BlockSpec((2, BLK, D), lambda i: (0, i, 0))


def _tc_layer0(x, wc, dp, mp):
    return pl.pallas_call(
        _tc_layer0_body,
        grid=(NB,),
        in_specs=[
            _spec_rows,
            pl.BlockSpec((D, 2 * D), lambda i: (0, 0)),
            _spec_dp,
            _spec_dp,
        ],
        out_specs=_spec_rows,
        out_shape=jax.ShapeDtypeStruct((N, D), jnp.float32),
    )(x, wc, dp, mp)


def _tc_act(p, wc, dp, mp, b):
    return pl.pallas_call(
        _tc_act_body,
        grid=(NB,),
        in_specs=[
            _spec_p,
            pl.BlockSpec((D, 2 * D), lambda i: (0, 0)),
            _spec_dp,
            _spec_dp,
            pl.BlockSpec((1, D), lambda i: (0, 0)),
        ],
        out_specs=_spec_rows,
        out_shape=jax.ShapeDtypeStruct((N, D), jnp.float32),
    )(p, wc, dp, mp, b)


def _tc_head(p, dp, b2, wm1, bm1, wm2p, bm2p):
    return pl.pallas_call(
        _tc_head_body,
        grid=(NB,),
        in_specs=[
            _spec_p,
            _spec_dp,
            pl.BlockSpec((1, D), lambda i: (0, 0)),
            pl.BlockSpec((D, 256), lambda i: (0, 0)),
            pl.BlockSpec((1, 256), lambda i: (0, 0)),
            pl.BlockSpec((256, 128), lambda i: (0, 0)),
            pl.BlockSpec((1, 128), lambda i: (0, 0)),
        ],
        out_specs=_spec_rows,
        out_shape=jax.ShapeDtypeStruct((N, D), jnp.float32),
    )(p, dp, b2, wm1, bm1, wm2p, bm2p)


def kernel(x, edge_index, id_index, edge_weight, W0, IW0, b0, W1, IW1, b1,
           W2, IW2, b2, Wm1, bm1, Wm2, bm2):
    rc4 = edge_index.reshape(2, NW, NCH, K).transpose(1, 2, 0, 3)
    ewb4 = jnp.broadcast_to(edge_weight[:, None], (E, L)).reshape(NW, NCH, K, L)
    ewq4 = jnp.broadcast_to(
        edge_weight[:, None], (E, L)).reshape(NW, NCH, K * L // D, D)
    idp = jnp.concatenate(
        [id_index, jnp.zeros((NID_PAD - NID,), jnp.int32)]).reshape(NW, 1, IPT)
    idvb3 = jnp.broadcast_to(
        ((jnp.arange(NID_PAD) < NID).astype(jnp.float32))[:, None],
        (NID_PAD, L)).reshape(NW, IPT, L)

    deg_p, mask_p = _sc_precompute(rc4, ewb4, idp, idvb3)

    wc0 = jnp.concatenate([W0, IW0], axis=1)
    wc1 = jnp.concatenate([W1, IW1], axis=1)
    wc2 = jnp.concatenate([W2, IW2], axis=1)
    b0r = b0.reshape(1, D)
    b1r = b1.reshape(1, D)
    b2r = b2.reshape(1, D)
    bm1r = bm1.reshape(1, 256)
    wm2p = jnp.zeros((256, 128), jnp.float32).at[:, :C].set(Wm2)
    bm2p = jnp.zeros((1, 128), jnp.float32).at[0, :C].set(bm2)

    s0 = _tc_layer0(x, wc0, deg_p, mask_p)
    p1 = _sc_agg(s0, rc4, ewq4)
    s1 = _tc_act(p1, wc1, deg_p, mask_p, b0r)
    p2 = _sc_agg(s1, rc4, ewq4)
    s2 = _tc_act(p2, wc2, deg_p, mask_p, b1r)
    p3 = _sc_agg(s2, rc4, ewq4)
    outp = _tc_head(p3, deg_p, b2r, Wm1, bm1r, wm2p, bm2p)
    return outp[:, :C]


# speedup vs baseline: 1.7180x; 1.0655x over previous
import functools

import jax
import jax.numpy as jnp
from jax import lax
from jax.experimental import pallas as pl
from jax.experimental.pallas import tpu as pltpu
from jax.experimental.pallas import tpu_sc as plsc

N = 10000
E = 320000
D = 128
H = 128
C = 6
NID = 1000

NC, NS, L = 2, 16, 16
NW = NC * NS
EPT = E // NW
K = 80
NCH = EPT // K
NPAD = 10240
NPT = NPAD // NS
NID_PAD = 1024
IPT = NID_PAD // NW

_mesh = plsc.VectorSubcoreMesh(
    core_axis_name="c", subcore_axis_name="s", num_cores=NC, num_subcores=NS
)


@functools.partial(
    pl.kernel,
    out_type=[
        jax.ShapeDtypeStruct((NC, NPAD, D), jnp.float32),
        jax.ShapeDtypeStruct((NC, NPAD, D), jnp.float32),
    ],
    mesh=_mesh,
    scratch_types=(
        [pltpu.VMEM((2, K), jnp.int32) for _ in range(3)]
        + [pltpu.VMEM((K * L // D, D), jnp.float32) for _ in range(3)]
        + [pltpu.VMEM((K, D), jnp.float32) for _ in range(3)]
        + [
            pltpu.VMEM((1, IPT), jnp.int32),
            pltpu.VMEM((IPT * L // D, D), jnp.float32),
            pltpu.VMEM((8, D), jnp.float32),
            pltpu.VMEM_SHARED((NPAD, D), jnp.float32),
        ]
        + [pltpu.SemaphoreType.DMA for _ in range(6)]
    ),
)
def _sc_precompute(rc4, ewq4, id3, idvq3, deg_out, mask_out, *refs):
    rc = refs[0:3]
    ewq = refs[3:6]
    wide = refs[6:9]
    id_v, idvq_v, zbuf, tab_sh = refs[9:13]
    esem = refs[13:16]
    ssem = refs[16:19]
    c = lax.axis_index("c")
    sid = lax.axis_index("s")
    wid = c * NS + sid
    for r in range(8):
        for j in range(D // L):
            zbuf[r, pl.ds(j * L, L)] = jnp.zeros((L,), jnp.float32)

    def zc(z, carry):
        pltpu.sync_copy(zbuf, tab_sh.at[pl.ds(sid * NPT + z * 8, 8)])
        return carry

    lax.fori_loop(0, NPT // 8, zc, 0)
    pltpu.sync_copy(id3.at[wid], id_v)
    pltpu.sync_copy(idvq3.at[wid], idvq_v)
    plsc.subcore_barrier()

    def issue(g, s):
        pltpu.sync_copy(rc4.at[wid, g], rc[s])
        pltpu.async_copy(ewq4.at[wid, g], ewq[s], esem[s])

    def drain(s):
        pltpu.make_async_copy(
            wide[s], tab_sh.at[rc[s].at[0]], ssem[s]).wait()

    def work_fire(g, s):
        pltpu.make_async_copy(ewq4.at[wid, g], ewq[s], esem[s]).wait()
        wv = wide[s]
        q = ewq[s]
        for e in range(K):
            w = q[e >> 3, pl.ds((e & 7) * L, L)]
            for j in range(D // L):
                wv[e, pl.ds(j * L, L)] = w
        pltpu.async_copy(wv, tab_sh.at[rc[s].at[0]], ssem[s], add=True)

    issue(0, 0)
    issue(1, 1)

    def pipe(i, carry):
        for b in range(3):
            g = 3 * i + b
            sp = (b - 1) % 3
            if b == 0:
                @pl.when(i >= 1)
                def _():
                    drain(sp)
            else:
                drain(sp)
            work_fire(g, b)
            issue(g + 2, (b + 2) % 3)
        return carry

    lax.fori_loop(0, (NCH - 2) // 3, pipe, 0)
    drain(2)
    work_fire(NCH - 2, 0)
    work_fire(NCH - 1, 1)
    drain(0)
    drain(1)
    plsc.subcore_barrier()
    pltpu.sync_copy(tab_sh.at[pl.ds(sid * NPT, NPT)],
                    deg_out.at[c, pl.ds(sid * NPT, NPT)])
    plsc.subcore_barrier()

    lax.fori_loop(0, NPT // 8, zc, 0)
    plsc.subcore_barrier()
    for e in range(IPT):
        w = idvq_v[e >> 3, pl.ds((e & 7) * L, L)]
        for j in range(D // L):
            wide[0][e, pl.ds(j * L, L)] = w
    pltpu.sync_copy(wide[0].at[pl.ds(0, IPT)], tab_sh.at[id_v.at[0]],
                    add=True)
    plsc.subcore_barrier()
    pltpu.sync_copy(tab_sh.at[pl.ds(sid * NPT, NPT)],
                    mask_out.at[c, pl.ds(sid * NPT, NPT)])


@functools.partial(
    pl.kernel,
    out_type=jax.ShapeDtypeStruct((NC, NPAD, D), jnp.float32),
    mesh=_mesh,
    scratch_types=(
        [pltpu.VMEM((2, K), jnp.int32) for _ in range(3)]
        + [pltpu.VMEM((K, D), jnp.float32) for _ in range(3)]
        + [pltpu.VMEM((K * L // D, D), jnp.float32) for _ in range(3)]
        + [
            pltpu.VMEM((8, D), jnp.float32),
            pltpu.VMEM_SHARED((NPAD, D), jnp.float32),
        ]
        + [pltpu.SemaphoreType.DMA for _ in range(9)]
    ),
)
def _sc_agg(s_hbm, rc4, ewq4, p_out, *refs):
    rc = refs[0:3]
    buf = refs[3:6]
    ewq = refs[6:9]
    zbuf, agg_sh = refs[9:11]
    gsem = refs[11:14]
    ssem = refs[14:17]
    esem = refs[17:20]
    c = lax.axis_index("c")
    sid = lax.axis_index("s")
    wid = c * NS + sid
    for r in range(8):
        for j in range(D // L):
            zbuf[r, pl.ds(j * L, L)] = jnp.zeros((L,), jnp.float32)

    def zc(z, carry):
        pltpu.sync_copy(zbuf,
                        agg_sh.at[pl.ds(sid * NPT + z * 8, 8)])
        return carry

    lax.fori_loop(0, NPT // 8, zc, 0)
    plsc.subcore_barrier()

    def issue(g, s):
        pltpu.sync_copy(rc4.at[wid, g], rc[s])
        pltpu.async_copy(s_hbm.at[rc[s].at[1]], buf[s], gsem[s])
        pltpu.async_copy(ewq4.at[wid, g], ewq[s], esem[s])

    def drain(s):
        pltpu.make_async_copy(
            buf[s], agg_sh.at[rc[s].at[0]], ssem[s]).wait()

    def work_fire(g, s):
        pltpu.make_async_copy(s_hbm.at[rc[s].at[1]], buf[s], gsem[s]).wait()
        pltpu.make_async_copy(ewq4.at[wid, g], ewq[s], esem[s]).wait()
        b = buf[s]
        q = ewq[s]
        for e in range(K):
            w = q[e >> 3, pl.ds((e & 7) * L, L)]
            for j in range(D // L):
                b[e, pl.ds(j * L, L)] = b[e, pl.ds(j * L, L)] * w
        pltpu.async_copy(b, agg_sh.at[rc[s].at[0]], ssem[s], add=True)

    issue(0, 0)
    issue(1, 1)

    def pipe(i, carry):
        for b in range(3):
            g = 3 * i + b
            sp = (b - 1) % 3
            if b == 0:
                @pl.when(i >= 1)
                def _():
                    drain(sp)
            else:
                drain(sp)
            work_fire(g, b)
            issue(g + 2, (b + 2) % 3)
        return carry

    lax.fori_loop(0, (NCH - 2) // 3, pipe, 0)
    drain(2)
    work_fire(NCH - 2, 0)
    work_fire(NCH - 1, 1)
    drain(0)
    drain(1)
    plsc.subcore_barrier()
    pltpu.sync_copy(agg_sh.at[pl.ds(sid * NPT, NPT)],
                    p_out.at[c, pl.ds(sid * NPT, NPT)])


BLK = 2000
NB = N // BLK


def _dinv_of(dp_ref):
    deg = dp_ref[0, :, 0] + dp_ref[1, :, 0]
    return jnp.where(deg > 0.0, lax.rsqrt(jnp.maximum(deg, 1e-12)), 0.0)


def _mask_of(mp_ref):
    return (mp_ref[0, :, 0] + mp_ref[1, :, 0]) > 0.0


def _tc_layer0_body(x_ref, wc_ref, dp_ref, mp_ref, o_ref):
    dinv = _dinv_of(dp_ref)
    m = _mask_of(mp_ref)
    t = jnp.dot(x_ref[...], wc_ref[...], preferred_element_type=jnp.float32)
    sel = jnp.where(m[:, None], t[:, D:], t[:, :D])
    o_ref[...] = sel * dinv[:, None]


def _tc_act_body(p_ref, wc_ref, dp_ref, mp_ref, b_ref, o_ref):
    dinv = _dinv_of(dp_ref)
    m = _mask_of(mp_ref)
    h = jax.nn.relu((p_ref[0] + p_ref[1]) * dinv[:, None] + b_ref[...])
    t = jnp.dot(h, wc_ref[...], preferred_element_type=jnp.float32)
    sel = jnp.where(m[:, None], t[:, D:], t[:, :D])
    o_ref[...] = sel * dinv[:, None]


def _tc_head_body(p_ref, dp_ref, b2_ref, wm1_ref, bm1_ref, wm2_ref, bm2_ref,
                  o_ref):
    dinv = _dinv_of(dp_ref)
    h = jax.nn.relu((p_ref[0] + p_ref[1]) * dinv[:, None] + b2_ref[...])
    z = jax.nn.relu(
        jnp.dot(h, wm1_ref[...], preferred_element_type=jnp.float32)
        + bm1_ref[...])
    o_ref[...] = (
        jnp.dot(z, wm2_ref[...], preferred_element_type=jnp.float32)
        + bm2_ref[...])


_spec_dp = pl.BlockSpec((2, BLK, D), lambda i: (0, i, 0))
_spec_rows = pl.BlockSpec((BLK, D), lambda i: (i, 0))
_spec_p = pl.BlockSpec((2, BLK, D), lambda i: (0, i, 0))


def _tc_layer0(x, wc, dp, mp):
    return pl.pallas_call(
        _tc_layer0_body,
        grid=(NB,),
        in_specs=[
            _spec_rows,
            pl.BlockSpec((D, 2 * D), lambda i: (0, 0)),
            _spec_dp,
            _spec_dp,
        ],
        out_specs=_spec_rows,
        out_shape=jax.ShapeDtypeStruct((N, D), jnp.float32),
    )(x, wc, dp, mp)


def _tc_act(p, wc, dp, mp, b):
    return pl.pallas_call(
        _tc_act_body,
        grid=(NB,),
        in_specs=[
            _spec_p,
            pl.BlockSpec((D, 2 * D), lambda i: (0, 0)),
            _spec_dp,
            _spec_dp,
            pl.BlockSpec((1, D), lambda i: (0, 0)),
        ],
        out_specs=_spec_rows,
        out_shape=jax.ShapeDtypeStruct((N, D), jnp.float32),
    )(p, wc, dp, mp, b)


def _tc_head(p, dp, b2, wm1, bm1, wm2p, bm2p):
    return pl.pallas_call(
        _tc_head_body,
        grid=(NB,),
        in_specs=[
            _spec_p,
            _spec_dp,
            pl.BlockSpec((1, D), lambda i: (0, 0)),
            pl.BlockSpec((D, 256), lambda i: (0, 0)),
            pl.BlockSpec((1, 256), lambda i: (0, 0)),
            pl.BlockSpec((256, 128), lambda i: (0, 0)),
            pl.BlockSpec((1, 128), lambda i: (0, 0)),
        ],
        out_specs=_spec_rows,
        out_shape=jax.ShapeDtypeStruct((N, D), jnp.float32),
    )(p, dp, b2, wm1, bm1, wm2p, bm2p)


def kernel(x, edge_index, id_index, edge_weight, W0, IW0, b0, W1, IW1, b1,
           W2, IW2, b2, Wm1, bm1, Wm2, bm2):
    rc4 = edge_index.reshape(2, NW, NCH, K).transpose(1, 2, 0, 3)
    ewq4 = jnp.broadcast_to(
        edge_weight[:, None], (E, L)).reshape(NW, NCH, K * L // D, D)
    idp = jnp.concatenate(
        [id_index, jnp.zeros((NID_PAD - NID,), jnp.int32)]).reshape(NW, 1, IPT)
    idvq3 = jnp.broadcast_to(
        ((jnp.arange(NID_PAD) < NID).astype(jnp.float32))[:, None],
        (NID_PAD, L)).reshape(NW, IPT * L // D, D)

    deg_p, mask_p = _sc_precompute(rc4, ewq4, idp, idvq3)

    wc0 = jnp.concatenate([W0, IW0], axis=1)
    wc1 = jnp.concatenate([W1, IW1], axis=1)
    wc2 = jnp.concatenate([W2, IW2], axis=1)
    b0r = b0.reshape(1, D)
    b1r = b1.reshape(1, D)
    b2r = b2.reshape(1, D)
    bm1r = bm1.reshape(1, 256)
    wm2p = jnp.zeros((256, 128), jnp.float32).at[:, :C].set(Wm2)
    bm2p = jnp.zeros((1, 128), jnp.float32).at[0, :C].set(bm2)

    s0 = _tc_layer0(x, wc0, deg_p, mask_p)
    p1 = _sc_agg(s0, rc4, ewq4)
    s1 = _tc_act(p1, wc1, deg_p, mask_p, b0r)
    p2 = _sc_agg(s1, rc4, ewq4)
    s2 = _tc_act(p2, wc2, deg_p, mask_p, b1r)
    p3 = _sc_agg(s2, rc4, ewq4)
    outp = _tc_head(p3, deg_p, b2r, Wm1, bm1r, wm2p, bm2p)
    return outp[:, :C]
